# baseline (speedup 1.0000x reference)
import functools

import jax
import jax.numpy as jnp
from jax import lax
from jax.experimental import pallas as pl
from jax.experimental.pallas import tpu as pltpu
from jax.experimental.pallas import tpu_sc as plsc

_NC = 2
_NS = 16
_L = 16
_NW = _NC * _NS

_BS, _N, _D = 16, 128, 64
_ROWS = _BS * _N
_RPW = _ROWS // _NW


def _sc_gather_body(msg_hbm, out_hbm, idx_v, rows_v, sem):
    wid = lax.axis_index("s") * _NC + lax.axis_index("c")
    base = wid * _RPW

    for k in range(_RPW // _L):
        r = base + k * _L + lax.iota(jnp.int32, _L)
        idx_v[pl.ds(k * _L, _L)] = (r << 7) + (r & (_N - 1))

    gather = pltpu.make_async_copy(msg_hbm.at[idx_v], rows_v, sem)
    gather.start()
    gather.wait()
    pltpu.sync_copy(rows_v, out_hbm.at[pl.ds(base, _RPW)])


@functools.cache
def _sc_gather():
    return pl.kernel(
        _sc_gather_body,
        out_type=jax.ShapeDtypeStruct((_ROWS, _D), jnp.float32),
        mesh=plsc.VectorSubcoreMesh(
            core_axis_name="c", subcore_axis_name="s",
            num_cores=_NC, num_subcores=_NS,
        ),
        scratch_types=[
            pltpu.VMEM((_RPW,), jnp.int32),
            pltpu.VMEM((_RPW, _D), jnp.float32),
            pltpu.SemaphoreType.DMA,
        ],
        compiler_params=pltpu.CompilerParams(use_tc_tiling_on_sc=False),
    )


_TC_ROWS = 256


def _tc_mask_body(g_ref, mm_ref, out_ref):
    num_msg = jnp.sum(mm_ref[...], axis=1, keepdims=True)
    out_ref[...] = jnp.where(num_msg != 0.0, g_ref[...], 0.0)


@jax.jit
def kernel(msg, msg_matrix):
    bs, n, _, d = msg.shape
    msg_flat = msg.reshape(bs * n * n, d)
    mm = msg_matrix.reshape(bs * n, n)

    gathered = _sc_gather()(msg_flat)

    out = pl.pallas_call(
        _tc_mask_body,
        grid=(_ROWS // _TC_ROWS,),
        in_specs=[
            pl.BlockSpec((_TC_ROWS, _D), lambda i: (i, 0)),
            pl.BlockSpec((_TC_ROWS, _N), lambda i: (i, 0)),
        ],
        out_specs=pl.BlockSpec((_TC_ROWS, _D), lambda i: (i, 0)),
        out_shape=jax.ShapeDtypeStruct((_ROWS, _D), jnp.float32),
    )(gathered, mm)

    return out.reshape(bs, n, d)


# speedup vs baseline: 1.5474x; 1.5474x over previous
import functools

import jax
import jax.numpy as jnp
from jax import lax
from jax.experimental import pallas as pl
from jax.experimental.pallas import tpu as pltpu
from jax.experimental.pallas import tpu_sc as plsc

_NC = 2
_NS = 16
_L = 16
_NW = _NC * _NS

_BS, _N, _D = 16, 128, 64
_ROWS = _BS * _N
_RPW = _ROWS // _NW


def _sc_gather_body(msg_hbm, out_hbm, rows_v, sem):
    wid = lax.axis_index("s") * _NC + lax.axis_index("c")
    base = wid * _RPW
    b = base >> 7
    i0 = base & (_N - 1)

    copies = [
        pltpu.make_async_copy(msg_hbm.at[b, i0 + k, i0 + k], rows_v.at[k], sem)
        for k in range(_RPW)
    ]
    for c in copies:
        c.start()
    for c in copies:
        c.wait()
    pltpu.sync_copy(rows_v, out_hbm.at[pl.ds(base, _RPW)])


@functools.cache
def _sc_gather():
    return pl.kernel(
        _sc_gather_body,
        out_type=jax.ShapeDtypeStruct((_ROWS, _D), jnp.float32),
        mesh=plsc.VectorSubcoreMesh(
            core_axis_name="c", subcore_axis_name="s",
            num_cores=_NC, num_subcores=_NS,
        ),
        scratch_types=[
            pltpu.VMEM((_RPW, _D), jnp.float32),
            pltpu.SemaphoreType.DMA,
        ],
    )


_TC_ROWS = 256


def _tc_mask_body(g_ref, mm_ref, out_ref):
    num_msg = jnp.sum(mm_ref[...], axis=1, keepdims=True)
    out_ref[...] = jnp.where(num_msg != 0.0, g_ref[...], 0.0)


@jax.jit
def kernel(msg, msg_matrix):
    bs, n, _, d = msg.shape
    mm = msg_matrix.reshape(bs * n, n)

    gathered = _sc_gather()(msg)

    out = pl.pallas_call(
        _tc_mask_body,
        grid=(_ROWS // _TC_ROWS,),
        in_specs=[
            pl.BlockSpec((_TC_ROWS, _D), lambda i: (i, 0)),
            pl.BlockSpec((_TC_ROWS, _N), lambda i: (i, 0)),
        ],
        out_specs=pl.BlockSpec((_TC_ROWS, _D), lambda i: (i, 0)),
        out_shape=jax.ShapeDtypeStruct((_ROWS, _D), jnp.float32),
    )(gathered, mm)

    return out.reshape(bs, n, d)


# speedup vs baseline: 6.6965x; 4.3276x over previous
import functools

import jax
import jax.numpy as jnp
from jax import lax
from jax.experimental import pallas as pl
from jax.experimental.pallas import tpu as pltpu
from jax.experimental.pallas import tpu_sc as plsc

_NC = 2
_NS = 16
_L = 16
_NW = _NC * _NS

_BS, _N, _D = 16, 128, 64
_ROWS = _BS * _N
_RPW = _ROWS // _NW
_T = 8


def _sc_body(msg_hbm, mm_hbm, out_hbm, t_v, mm_v, accs_v, idx_v, sem):
    wid = lax.axis_index("s") * _NC + lax.axis_index("c")
    base = wid * _RPW
    b = base >> 7
    i0 = pl.multiple_of(base & (_N - 1), _RPW)

    stride = _N * _D + 1

    step = lax.iota(jnp.int32, _L) * stride

    def idx_body(q, carry):
        c0 = (base * _D + q) * _N + i0
        for c in range(_RPW // _L):
            idx_v[q, pl.ds(c * _L, _L)] = c0 + c * _L * stride + step
        return carry

    lax.fori_loop(0, _D, idx_body, 0)

    def fire_body(q, carry):
        pltpu.make_async_copy(
            msg_hbm.at[idx_v.at[q]], t_v.at[q // _T, q % _T], sem
        ).start()
        return carry

    lax.fori_loop(0, _D, fire_body, 0)

    pltpu.sync_copy(mm_hbm.at[b, pl.ds(i0 // _T, _T)], mm_v)

    def row_body(k, carry):
        acc = mm_v[k // _T, k % _T, pl.ds(0, _L)]
        for c in range(1, _N // _L):
            acc = acc + mm_v[k // _T, k % _T, pl.ds(c * _L, _L)]
        accs_v[k, :] = acc
        return carry

    lax.fori_loop(0, _RPW, row_body, 0)

    lane = lax.iota(jnp.int32, _L)

    def mask_body(k, ms4):
        vec = accs_v[k, :]
        s = vec[0]
        for t in range(1, _L):
            s = s + vec[t]
        ms = jnp.where(s != 0.0, jnp.float32(1.0), jnp.float32(0.0))
        h = jnp.where(lane == (k & (_L - 1)), jnp.float32(1.0),
                      jnp.float32(0.0))
        c = k >> 4
        return tuple(
            m + (h * jnp.where(c == cc, jnp.float32(1.0), jnp.float32(0.0)))
            * (ms - m)
            for cc, m in enumerate(ms4)
        )

    zeros = jnp.zeros((_L,), jnp.float32)
    masks = lax.fori_loop(
        0, _RPW, mask_body, (zeros, zeros, zeros, zeros)
    )

    def drain_body(q, carry):
        pltpu.make_async_copy(
            msg_hbm.at[idx_v.at[q]], t_v.at[q // _T, q % _T], sem
        ).wait()
        return carry

    lax.fori_loop(0, _D, drain_body, 0)

    def scale_body(q, carry):
        for c in range(_RPW // _L):
            t_v[q // _T, q % _T, pl.ds(c * _L, _L)] = (
                t_v[q // _T, q % _T, pl.ds(c * _L, _L)] * masks[c]
            )
        return carry

    lax.fori_loop(0, _D, scale_body, 0)

    pltpu.sync_copy(t_v, out_hbm.at[b, :, :, pl.ds(i0, _RPW)])


@functools.cache
def _sc_call():
    return pl.kernel(
        _sc_body,
        out_type=jax.ShapeDtypeStruct((_BS, _T, _T, _N), jnp.float32),
        mesh=plsc.VectorSubcoreMesh(
            core_axis_name="c", subcore_axis_name="s",
            num_cores=_NC, num_subcores=_NS,
        ),
        scratch_types=[
            pltpu.VMEM((_T, _T, _RPW), jnp.float32),
            pltpu.VMEM((_T, _T, _N), jnp.float32),
            pltpu.VMEM((_RPW, _L), jnp.float32),
            pltpu.VMEM((_D, _RPW), jnp.int32),
            pltpu.SemaphoreType.DMA,
        ],
        compiler_params=pltpu.CompilerParams(use_tc_tiling_on_sc=False),
    )


@jax.jit
def kernel(msg, msg_matrix):
    bs, n, _, d = msg.shape
    msg_flat = jnp.transpose(msg, (0, 1, 3, 2)).reshape(-1)
    mm4 = msg_matrix.reshape(bs, n // _T, _T, n)
    out5 = _sc_call()(msg_flat, mm4)
    return jnp.transpose(out5.reshape(bs, d, n), (0, 2, 1))


# speedup vs baseline: 6.7237x; 1.0041x over previous
import functools

import jax
import jax.numpy as jnp
from jax import lax
from jax.experimental import pallas as pl
from jax.experimental.pallas import tpu as pltpu
from jax.experimental.pallas import tpu_sc as plsc

_NC = 2
_NS = 16
_L = 16
_NW = _NC * _NS

_BS, _N, _D = 16, 128, 64
_ROWS = _BS * _N
_RPW = _ROWS // _NW
_T = 8


def _sc_body(msg_hbm, mm_hbm, out_hbm, t_v, mm_v, accs_v, idx_v, sem):
    wid = lax.axis_index("s") * _NC + lax.axis_index("c")
    base = wid * _RPW
    b = base >> 7
    i0 = pl.multiple_of(base & (_N - 1), _RPW)

    stride = _N * _D + 1

    step = lax.iota(jnp.int32, _L) * stride

    def idx_body(q, carry):
        c0 = (base * _D + q) * _N + i0
        for c in range(_RPW // _L):
            idx_v[q, pl.ds(c * _L, _L)] = c0 + c * _L * stride + step
        return carry

    lax.fori_loop(0, _D, idx_body, 0)

    def fire_body(q, carry):
        pltpu.make_async_copy(
            msg_hbm.at[idx_v.at[q]], t_v.at[q // _T, q % _T], sem
        ).start()
        return carry

    lax.fori_loop(0, _D, fire_body, 0)

    pltpu.sync_copy(mm_hbm.at[b, pl.ds(i0 // _T, _T)], mm_v)

    def row_body(k, carry):
        acc = mm_v[k // _T, k % _T, pl.ds(0, _L)]
        for c in range(1, _N // _L):
            acc = acc + mm_v[k // _T, k % _T, pl.ds(c * _L, _L)]
        accs_v[k, :] = acc
        return carry

    lax.fori_loop(0, _RPW, row_body, 0)

    rows16 = lax.iota(jnp.int32, _L)
    masks = []
    for c in range(_RPW // _L):
        ridx = c * _L + rows16
        s = plsc.load_gather(accs_v, [ridx, jnp.zeros((_L,), jnp.int32)])
        for l in range(1, _L):
            s = s + plsc.load_gather(
                accs_v, [ridx, jnp.full((_L,), l, jnp.int32)]
            )
        masks.append(
            jnp.where(s != 0.0, jnp.float32(1.0), jnp.float32(0.0))
        )

    def drain_body(q, carry):
        pltpu.make_async_copy(
            msg_hbm.at[idx_v.at[q]], t_v.at[q // _T, q % _T], sem
        ).wait()
        return carry

    lax.fori_loop(0, _D, drain_body, 0)

    def scale_body(q, carry):
        for c in range(_RPW // _L):
            t_v[q // _T, q % _T, pl.ds(c * _L, _L)] = (
                t_v[q // _T, q % _T, pl.ds(c * _L, _L)] * masks[c]
            )
        return carry

    lax.fori_loop(0, _D, scale_body, 0)

    pltpu.sync_copy(t_v, out_hbm.at[b, :, :, pl.ds(i0, _RPW)])


@functools.cache
def _sc_call():
    return pl.kernel(
        _sc_body,
        out_type=jax.ShapeDtypeStruct((_BS, _T, _T, _N), jnp.float32),
        mesh=plsc.VectorSubcoreMesh(
            core_axis_name="c", subcore_axis_name="s",
            num_cores=_NC, num_subcores=_NS,
        ),
        scratch_types=[
            pltpu.VMEM((_T, _T, _RPW), jnp.float32),
            pltpu.VMEM((_T, _T, _N), jnp.float32),
            pltpu.VMEM((_RPW, _L), jnp.float32),
            pltpu.VMEM((_D, _RPW), jnp.int32),
            pltpu.SemaphoreType.DMA,
        ],
        compiler_params=pltpu.CompilerParams(
            use_tc_tiling_on_sc=False, needs_layout_passes=False
        ),
    )


@jax.jit
def kernel(msg, msg_matrix):
    bs, n, _, d = msg.shape
    msg_flat = jnp.transpose(msg, (0, 1, 3, 2)).reshape(-1)
    mm4 = msg_matrix.reshape(bs, n // _T, _T, n)
    out5 = _sc_call()(msg_flat, mm4)
    return jnp.transpose(out5.reshape(bs, d, n), (0, 2, 1))
